# baseline (speedup 1.0000x reference)
import numpy as np
import jax
import jax.numpy as jnp
from jax.experimental import pallas as pl
from jax.experimental.pallas import tpu as pltpu

_GRID = 10
_N = _GRID * _GRID
_NP = 104
_HID = 128
_LAYERS = 6
_NODE_F = 5


def _build_adjacency():
    ah = np.zeros((_NP, _NP), np.float32)
    av = np.zeros((_NP, _NP), np.float32)
    for r in range(_GRID):
        for c in range(_GRID):
            n = r * _GRID + c
            if c + 1 < _GRID:
                ah[n, n + 1] = 1.0
                ah[n + 1, n] = 1.0
            if r + 1 < _GRID:
                av[n, n + _GRID] = 1.0
                av[n + _GRID, n] = 1.0
    return ah, av

_AH_NP, _AV_NP = _build_adjacency()


def _gnn_kernel(x_ref, ah_ref, av_ref, encW_ref, encb_ref, mw1_ref,
                wedge_ref, mb1_ref, mw2ub_ref, b2ub_ref, ua_ref, ub1_ref,
                uw2_ref, ub2_ref, g_ref, lb_ref, dw1_ref, db1_ref, dw2_ref,
                db2_ref, out_ref):
    f32 = jnp.float32
    xb = x_ref[...]
    h = jnp.maximum(
        jnp.dot(xb, encW_ref[...], preferred_element_type=f32) + encb_ref[...], 0.0)
    m = h.shape[0]
    bb = m // _NP
    ah = ah_ref[...]
    av = av_ref[...]
    row = jax.lax.broadcasted_iota(jnp.int32, (m, _HID), 0)
    n = row % _NP
    c = n % _GRID
    r = n // _GRID
    valid = (n < _N).astype(f32)
    deg = ((c != 0).astype(f32) + (c != _GRID - 1).astype(f32) +
           (r != 0).astype(f32) + (r != _GRID - 1).astype(f32)) * valid

    def to_nodes_minor(a):
        return jnp.transpose(a.reshape(bb, _NP, _HID), (0, 2, 1)).reshape(
            bb * _HID, _NP)

    def to_feats_minor(z):
        return jnp.transpose(z.reshape(bb, _HID, _NP), (0, 2, 1)).reshape(
            bb * _NP, _HID)

    for l in range(_LAYERS):
        y = jnp.dot(h, mw1_ref[l], preferred_element_type=f32) + mb1_ref[l]
        a0 = jnp.maximum(y, 0.0)
        a1 = jnp.maximum(y + wedge_ref[l], 0.0)
        z = (jnp.dot(to_nodes_minor(a0), ah, preferred_element_type=f32) +
             jnp.dot(to_nodes_minor(a1), av, preferred_element_type=f32))
        agg0 = to_feats_minor(z)
        upre = (jnp.dot(h, ua_ref[l], preferred_element_type=f32) +
                jnp.dot(agg0, mw2ub_ref[l], preferred_element_type=f32) +
                deg * b2ub_ref[l] + ub1_ref[l])
        u = (jnp.dot(jnp.maximum(upre, 0.0), uw2_ref[l],
                     preferred_element_type=f32) + ub2_ref[l])
        pre = h + u
        mu = jnp.mean(pre, axis=1, keepdims=True)
        var = jnp.mean((pre - mu) * (pre - mu), axis=1, keepdims=True)
        h = (pre - mu) * jax.lax.rsqrt(var + 1e-5) * g_ref[l] + lb_ref[l]
    d1 = jnp.maximum(
        jnp.dot(h, dw1_ref[...], preferred_element_type=f32) + db1_ref[...], 0.0)
    out_ref[...] = (jnp.dot(d1, dw2_ref[...], preferred_element_type=f32) +
                    db2_ref[...])


def kernel(x, enc_W, enc_b, msg_W1, msg_b1, msg_W2, msg_b2,
           upd_W1, upd_b1, upd_W2, upd_b2, ln_g, ln_b,
           dec_W1, dec_b1, dec_W2, dec_b2):
    B = x.shape[0]
    BB = 64
    M_BLK = BB * _NP
    xp = jnp.pad(x, ((0, 0), (0, _NP - _N), (0, 0)))
    x2 = xp.reshape(B * _NP, _NODE_F)

    mw1 = msg_W1[:, :_HID, :]
    wedge = msg_W1[:, _HID:, :]
    ua = upd_W1[:, :_HID, :]
    ub = upd_W1[:, _HID:, :]
    mw2ub = jnp.einsum('lij,ljk->lik', msg_W2, ub)
    b2ub = jnp.einsum('lj,ljk->lk', msg_b2, ub)
    ah = jnp.asarray(_AH_NP)
    av = jnp.asarray(_AV_NP)
    r2 = lambda a: a.reshape(1, -1)
    r3 = lambda a: a.reshape(_LAYERS, 1, -1)

    full = lambda a: pl.BlockSpec(a.shape, lambda i: (0,) * a.ndim)
    operands = (x2, ah, av, enc_W, r2(enc_b), mw1, wedge, r3(msg_b1), mw2ub,
                r3(b2ub), ua, r3(upd_b1), upd_W2, r3(upd_b2),
                r3(ln_g), r3(ln_b), dec_W1, r2(dec_b1), dec_W2, r2(dec_b2))
    in_specs = [pl.BlockSpec((M_BLK, _NODE_F), lambda i: (i, 0))]
    in_specs += [full(a) for a in operands[1:]]

    out = pl.pallas_call(
        _gnn_kernel,
        grid=(B // BB,),
        in_specs=in_specs,
        out_specs=pl.BlockSpec((M_BLK, 1), lambda i: (i, 0)),
        out_shape=jax.ShapeDtypeStruct((B * _NP, 1), jnp.float32),
        compiler_params=pltpu.CompilerParams(
            dimension_semantics=("parallel",)),
    )(*operands)
    return out.reshape(B, _NP)[:, :_N]


# speedup vs baseline: 1.5234x; 1.5234x over previous
import jax
import jax.numpy as jnp
from jax.experimental import pallas as pl
from jax.experimental.pallas import tpu as pltpu

_GRID = 10
_N = _GRID * _GRID
_HID = 128
_LAYERS = 6
_NODE_F = 5


def _gnn_kernel(x_ref, encW_ref, encb_ref, mw1_ref, wedge_ref, mb1_ref,
                mw2_ref, mb2_ref, ua_ref, ub_ref, ub1_ref, uw2_ref, ub2_ref,
                g_ref, lb_ref, dw1_ref, db1_ref, dw2_ref, db2_ref, out_ref):
    f32 = jnp.float32
    xb = x_ref[...]
    h = jnp.maximum(
        jnp.dot(xb, encW_ref[...], preferred_element_type=f32) + encb_ref[...], 0.0)
    m = h.shape[0]
    row = jax.lax.broadcasted_iota(jnp.int32, (m, _HID), 0)
    n = row % _N
    c = n % _GRID
    mask_l = (c != 0).astype(f32)
    mask_r = (c != _GRID - 1).astype(f32)
    mask_u = (n >= _GRID).astype(f32)
    mask_d = (n < _N - _GRID).astype(f32)
    deg = mask_l + mask_r + mask_u + mask_d
    for l in range(_LAYERS):
        y = jnp.dot(h, mw1_ref[l], preferred_element_type=f32) + mb1_ref[l]
        a0 = jnp.maximum(y, 0.0)
        a1 = jnp.maximum(y + wedge_ref[l], 0.0)
        agg0 = (mask_l * pltpu.roll(a0, 1, 0) +
                mask_r * pltpu.roll(a0, m - 1, 0) +
                mask_u * pltpu.roll(a1, _GRID, 0) +
                mask_d * pltpu.roll(a1, m - _GRID, 0))
        agg = (jnp.dot(agg0, mw2_ref[l], preferred_element_type=f32) +
               deg * mb2_ref[l])
        upre = (jnp.dot(h, ua_ref[l], preferred_element_type=f32) +
                jnp.dot(agg, ub_ref[l], preferred_element_type=f32) +
                ub1_ref[l])
        u = (jnp.dot(jnp.maximum(upre, 0.0), uw2_ref[l],
                     preferred_element_type=f32) + ub2_ref[l])
        pre = h + u
        mu = jnp.mean(pre, axis=1, keepdims=True)
        var = jnp.mean((pre - mu) * (pre - mu), axis=1, keepdims=True)
        h = (pre - mu) * jax.lax.rsqrt(var + 1e-5) * g_ref[l] + lb_ref[l]
    d1 = jnp.maximum(
        jnp.dot(h, dw1_ref[...], preferred_element_type=f32) + db1_ref[...], 0.0)
    out_ref[...] = (jnp.dot(d1, dw2_ref[...], preferred_element_type=f32) +
                    db2_ref[...])


def kernel(x, enc_W, enc_b, msg_W1, msg_b1, msg_W2, msg_b2,
           upd_W1, upd_b1, upd_W2, upd_b2, ln_g, ln_b,
           dec_W1, dec_b1, dec_W2, dec_b2):
    B = x.shape[0]
    BB = 32
    M_BLK = BB * _N
    x2 = x.reshape(B * _N, _NODE_F)

    mw1 = msg_W1[:, :_HID, :]
    wedge = msg_W1[:, _HID:, :]
    ua = upd_W1[:, :_HID, :]
    ub = upd_W1[:, _HID:, :]
    r2 = lambda a: a.reshape(1, -1)
    r3 = lambda a: a.reshape(_LAYERS, 1, -1)

    full = lambda a: pl.BlockSpec(a.shape, lambda i: (0,) * a.ndim)
    operands = (x2, enc_W, r2(enc_b), mw1, wedge, r3(msg_b1), msg_W2,
                r3(msg_b2), ua, ub, r3(upd_b1), upd_W2, r3(upd_b2),
                r3(ln_g), r3(ln_b), dec_W1, r2(dec_b1), dec_W2, r2(dec_b2))
    in_specs = [pl.BlockSpec((M_BLK, _NODE_F), lambda i: (i, 0))]
    in_specs += [full(a) for a in operands[1:]]

    out = pl.pallas_call(
        _gnn_kernel,
        grid=(B // BB,),
        in_specs=in_specs,
        out_specs=pl.BlockSpec((M_BLK, 1), lambda i: (i, 0)),
        out_shape=jax.ShapeDtypeStruct((B * _N, 1), jnp.float32),
        compiler_params=pltpu.CompilerParams(
            dimension_semantics=("parallel",)),
    )(*operands)
    return out.reshape(B, _N)


# speedup vs baseline: 1.6187x; 1.0625x over previous
import jax
import jax.numpy as jnp
from jax.experimental import pallas as pl
from jax.experimental.pallas import tpu as pltpu

_GRID = 10
_N = _GRID * _GRID
_HID = 128
_LAYERS = 6
_NODE_F = 5


def _gnn_kernel(x_ref, encW_ref, encb_ref, mw1_ref, wedge_ref, mb1_ref,
                mw2_ref, mb2_ref, ua_ref, ub_ref, ub1_ref, uw2_ref, ub2_ref,
                g_ref, lb_ref, dw1_ref, db1_ref, dw2_ref, db2_ref, out_ref):
    f32 = jnp.float32
    xb = x_ref[...]
    h = jnp.maximum(
        jnp.dot(xb, encW_ref[...], preferred_element_type=f32) + encb_ref[...], 0.0)
    m = h.shape[0]
    row = jax.lax.broadcasted_iota(jnp.int32, (m, _HID), 0)
    n = row % _N
    c = n % _GRID
    mask_l = (c != 0).astype(f32)
    mask_r = (c != _GRID - 1).astype(f32)
    mask_u = (n >= _GRID).astype(f32)
    mask_d = (n < _N - _GRID).astype(f32)
    deg = mask_l + mask_r + mask_u + mask_d
    for l in range(_LAYERS):
        y = jnp.dot(h, mw1_ref[l], preferred_element_type=f32) + mb1_ref[l]
        a0 = jnp.maximum(y, 0.0)
        a1 = jnp.maximum(y + wedge_ref[l], 0.0)
        agg0 = (mask_l * jnp.roll(a0, 1, 0) +
                mask_r * jnp.roll(a0, -1, 0) +
                mask_u * jnp.roll(a1, _GRID, 0) +
                mask_d * jnp.roll(a1, -_GRID, 0))
        agg = (jnp.dot(agg0, mw2_ref[l], preferred_element_type=f32) +
               deg * mb2_ref[l])
        upre = (jnp.dot(h, ua_ref[l], preferred_element_type=f32) +
                jnp.dot(agg, ub_ref[l], preferred_element_type=f32) +
                ub1_ref[l])
        u = (jnp.dot(jnp.maximum(upre, 0.0), uw2_ref[l],
                     preferred_element_type=f32) + ub2_ref[l])
        pre = h + u
        mu = jnp.mean(pre, axis=1, keepdims=True)
        var = jnp.mean((pre - mu) * (pre - mu), axis=1, keepdims=True)
        h = (pre - mu) * jax.lax.rsqrt(var + 1e-5) * g_ref[l] + lb_ref[l]
    d1 = jnp.maximum(
        jnp.dot(h, dw1_ref[...], preferred_element_type=f32) + db1_ref[...], 0.0)
    out_ref[...] = (jnp.dot(d1, dw2_ref[...], preferred_element_type=f32) +
                    db2_ref[...])


def kernel(x, enc_W, enc_b, msg_W1, msg_b1, msg_W2, msg_b2,
           upd_W1, upd_b1, upd_W2, upd_b2, ln_g, ln_b,
           dec_W1, dec_b1, dec_W2, dec_b2):
    B = x.shape[0]
    BB = 64
    M_BLK = BB * _N
    x2 = x.reshape(B * _N, _NODE_F)

    mw1 = msg_W1[:, :_HID, :]
    wedge = msg_W1[:, _HID:, :]
    ua = upd_W1[:, :_HID, :]
    ub = upd_W1[:, _HID:, :]
    r2 = lambda a: a.reshape(1, -1)
    r3 = lambda a: a.reshape(_LAYERS, 1, -1)

    full = lambda a: pl.BlockSpec(a.shape, lambda i: (0,) * a.ndim)
    operands = (x2, enc_W, r2(enc_b), mw1, wedge, r3(msg_b1), msg_W2,
                r3(msg_b2), ua, ub, r3(upd_b1), upd_W2, r3(upd_b2),
                r3(ln_g), r3(ln_b), dec_W1, r2(dec_b1), dec_W2, r2(dec_b2))
    in_specs = [pl.BlockSpec((M_BLK, _NODE_F), lambda i: (i, 0))]
    in_specs += [full(a) for a in operands[1:]]

    out = pl.pallas_call(
        _gnn_kernel,
        grid=(B // BB,),
        in_specs=in_specs,
        out_specs=pl.BlockSpec((M_BLK, 1), lambda i: (i, 0)),
        out_shape=jax.ShapeDtypeStruct((B * _N, 1), jnp.float32),
        compiler_params=pltpu.CompilerParams(
            dimension_semantics=("parallel",)),
    )(*operands)
    return out.reshape(B, _N)


# speedup vs baseline: 1.6212x; 1.0015x over previous
import jax
import jax.numpy as jnp
from jax.experimental import pallas as pl
from jax.experimental.pallas import tpu as pltpu

_GRID = 10
_N = _GRID * _GRID
_HID = 128
_LAYERS = 6
_NODE_F = 5


def _gnn_kernel(x_ref, encW_ref, encb_ref, mw1_ref, wedge_ref, mb1_ref,
                mw2_ref, mb2_ref, ua_ref, ub_ref, ub1_ref, uw2_ref, ub2_ref,
                g_ref, lb_ref, dw1_ref, db1_ref, dw2_ref, db2_ref, out_ref):
    f32 = jnp.float32
    xb = x_ref[...]
    h = jnp.maximum(
        jnp.dot(xb, encW_ref[...], preferred_element_type=f32) + encb_ref[...], 0.0)
    m = h.shape[0]
    row = jax.lax.broadcasted_iota(jnp.int32, (m, _HID), 0)
    n = row % _N
    c = n % _GRID
    mask_l = (c != 0).astype(f32)
    mask_r = (c != _GRID - 1).astype(f32)
    mask_u = (n >= _GRID).astype(f32)
    mask_d = (n < _N - _GRID).astype(f32)
    deg = mask_l + mask_r + mask_u + mask_d
    for l in range(_LAYERS):
        y = jnp.dot(h, mw1_ref[l], preferred_element_type=f32) + mb1_ref[l]
        a0 = jnp.maximum(y, 0.0)
        a1 = jnp.maximum(y + wedge_ref[l], 0.0)
        agg0 = (mask_l * pltpu.roll(a0, 1, 0) +
                mask_r * pltpu.roll(a0, m - 1, 0) +
                mask_u * pltpu.roll(a1, _GRID, 0) +
                mask_d * pltpu.roll(a1, m - _GRID, 0))
        agg = (jnp.dot(agg0, mw2_ref[l], preferred_element_type=f32) +
               deg * mb2_ref[l])
        upre = (jnp.dot(h, ua_ref[l], preferred_element_type=f32) +
                jnp.dot(agg, ub_ref[l], preferred_element_type=f32) +
                ub1_ref[l])
        u = (jnp.dot(jnp.maximum(upre, 0.0), uw2_ref[l],
                     preferred_element_type=f32) + ub2_ref[l])
        pre = h + u
        mu = jnp.mean(pre, axis=1, keepdims=True)
        var = jnp.mean((pre - mu) * (pre - mu), axis=1, keepdims=True)
        h = (pre - mu) * jax.lax.rsqrt(var + 1e-5) * g_ref[l] + lb_ref[l]
    d1 = jnp.maximum(
        jnp.dot(h, dw1_ref[...], preferred_element_type=f32) + db1_ref[...], 0.0)
    out_ref[...] = (jnp.dot(d1, dw2_ref[...], preferred_element_type=f32) +
                    db2_ref[...])


def kernel(x, enc_W, enc_b, msg_W1, msg_b1, msg_W2, msg_b2,
           upd_W1, upd_b1, upd_W2, upd_b2, ln_g, ln_b,
           dec_W1, dec_b1, dec_W2, dec_b2):
    B = x.shape[0]
    BB = 64
    M_BLK = BB * _N
    x2 = x.reshape(B * _N, _NODE_F)

    mw1 = msg_W1[:, :_HID, :]
    wedge = msg_W1[:, _HID:, :]
    ua = upd_W1[:, :_HID, :]
    ub = upd_W1[:, _HID:, :]
    r2 = lambda a: a.reshape(1, -1)
    r3 = lambda a: a.reshape(_LAYERS, 1, -1)

    full = lambda a: pl.BlockSpec(a.shape, lambda i: (0,) * a.ndim)
    operands = (x2, enc_W, r2(enc_b), mw1, wedge, r3(msg_b1), msg_W2,
                r3(msg_b2), ua, ub, r3(upd_b1), upd_W2, r3(upd_b2),
                r3(ln_g), r3(ln_b), dec_W1, r2(dec_b1), dec_W2, r2(dec_b2))
    in_specs = [pl.BlockSpec((M_BLK, _NODE_F), lambda i: (i, 0))]
    in_specs += [full(a) for a in operands[1:]]

    out = pl.pallas_call(
        _gnn_kernel,
        grid=(B // BB,),
        in_specs=in_specs,
        out_specs=pl.BlockSpec((M_BLK, 1), lambda i: (i, 0)),
        out_shape=jax.ShapeDtypeStruct((B * _N, 1), jnp.float32),
        compiler_params=pltpu.CompilerParams(
            dimension_semantics=("arbitrary",)),
    )(*operands)
    return out.reshape(B, _N)
